# Initial kernel scaffold; baseline (speedup 1.0000x reference)
import jax
import jax.numpy as jnp
from jax.experimental import pallas as pl


def kernel(x_f):
    raise NotImplementedError("write your pallas kernel here")


# speedup vs baseline: 18.7089x; 18.7089x over previous
import jax
import jax.numpy as jnp
from jax.experimental import pallas as pl

_N = 5000
_K = 4
_RADIUS = 0.02
_ROWS = 256
_PAD = 5120
_INF = jnp.inf


def _knn_body(xr_ref, xc_ref, zn_ref, invs_ref):
    pid = pl.program_id(0)
    xr = xr_ref[...]
    xc = xc_ref[...]
    d = jnp.abs(xr - xc)
    col = jax.lax.broadcasted_iota(jnp.int32, (_ROWS, _PAD), 1)
    row = pid * _ROWS + jax.lax.broadcasted_iota(jnp.int32, (_ROWS, _PAD), 0)
    d = jnp.where(col == row, _INF, d)
    d = jnp.where(d > _RADIUS, _INF, d)
    rowvec = pid * _ROWS + jax.lax.broadcasted_iota(jnp.int32, (_ROWS, 1), 0)
    acc = jnp.zeros((_ROWS, 1), jnp.float32)
    for k in range(_K):
        m = jnp.min(d, axis=1, keepdims=True)
        am = jnp.min(jnp.where(d == m, col, _PAD), axis=1, keepdims=True)
        valid = m <= _RADIUS
        zn_ref[:, k : k + 1] = jnp.where(valid, am, rowvec)
        acc = acc + jnp.where(valid, m * m, 0.0)
        if k < _K - 1:
            d = jnp.where(col == am, _INF, d)
    invs_ref[...] = 1.0 / (acc + 1e-8)


def kernel(x_f):
    x = x_f.reshape(-1)
    xp = jnp.full((_PAD,), 9.0, jnp.float32).at[: _N].set(x)
    zn, invs = pl.pallas_call(
        _knn_body,
        grid=(_PAD // _ROWS,),
        in_specs=[
            pl.BlockSpec((_ROWS, 1), lambda i: (i, 0)),
            pl.BlockSpec((1, _PAD), lambda i: (0, 0)),
        ],
        out_specs=[
            pl.BlockSpec((_ROWS, _K), lambda i: (i, 0)),
            pl.BlockSpec((_ROWS, 1), lambda i: (i, 0)),
        ],
        out_shape=[
            jax.ShapeDtypeStruct((_PAD, _K), jnp.int32),
            jax.ShapeDtypeStruct((_PAD, 1), jnp.float32),
        ],
    )(xp.reshape(_PAD, 1), xp.reshape(1, _PAD))
    return invs[:_N].reshape(_N, 1, 1), zn[:_N]


# speedup vs baseline: 21.4741x; 1.1478x over previous
import jax
import jax.numpy as jnp
import numpy as np
from jax import lax
from jax.experimental import pallas as pl
from jax.experimental.pallas import tpu as pltpu
from jax.experimental.pallas import tpu_sc as plsc

_N = 5000
_K = 4
_RF = np.float32(0.02)
_MARGIN = np.float32(0.0205)
_PAD = 5120
_XPAD = _PAD + 32
_NB = 256
_NC = 2
_NS = 16
_L = 16
_CHUNK = _PAD // _NS
_QCHUNK = _PAD // (_NC * _NS)
_INF = np.float32(np.inf)
_BIGI = np.int32(2147483647)


def _bin_of(v):
    return jnp.minimum((v * 256.0).astype(jnp.int32), 255)


def _body(x_hbm, zn0_hbm, zn1_hbm, zn2_hbm, zn3_hbm, invs_hbm,
          xchunk_v, origchunk_v, bins_v, hist_v, hists_all_v, tot_v,
          start_v, base_v, pos_v, xs_v, orig_v,
          zn0_v, zn1_v, zn2_v, zn3_v, invs_v, oidx_v,
          hists_sh, xs_sh, orig_sh):
    cid = lax.axis_index("c")
    sid = lax.axis_index("s")
    cbase = pl.multiple_of(sid * _CHUNK, _CHUNK)
    qbase = pl.multiple_of((cid * _NS + sid) * _QCHUNK, _QCHUNK)
    lane = lax.iota(jnp.int32, _L)
    oh16 = jnp.where(lane == 0, 1, 0).astype(jnp.int32)
    zero16 = jnp.zeros((_L,), jnp.int32)

    pltpu.sync_copy(x_hbm.at[pl.ds(cbase, _CHUNK)], xchunk_v)
    for j in range(_CHUNK // _L):
        xv = xchunk_v[pl.ds(j * _L, _L)]
        bins_v[pl.ds(j * _L, _L)] = _bin_of(xv)
        origchunk_v[pl.ds(j * _L, _L)] = cbase + j * _L + lane
    for j in range(_NB // _L):
        hist_v[pl.ds(j * _L, _L)] = zero16

    def hist_step(j, c):
        bv = bins_v[pl.ds(pl.multiple_of(j * _L, _L), _L)]
        for l in range(_L):
            b = bv[l]
            hv = hist_v[pl.ds(b, _L)]
            hist_v[pl.ds(b, _L)] = hv + oh16
        return c

    lax.fori_loop(0, _CHUNK // _L, hist_step, jnp.int32(0))

    pltpu.sync_copy(hist_v.at[pl.ds(0, _NB)], hists_sh.at[sid])
    plsc.subcore_barrier()
    pltpu.sync_copy(hists_sh, hists_all_v)
    for j in range(_NB // _L):
        sl = pl.ds(j * _L, _L)

        def acc_step(s, t):
            return t + hists_all_v[s, sl]

        tot_v[sl] = lax.fori_loop(0, _NS, acc_step, zero16)
        base_v[sl] = lax.fori_loop(0, sid, acc_step, zero16)

    carry = jnp.int32(0)
    for j in range(_NB // _L):
        sl = pl.ds(j * _L, _L)
        tchunk = tot_v[sl]
        pv = zero16
        for l in range(_L):
            pv = jnp.where(lane == l, carry, pv)
            carry = carry + tchunk[l]
        start_v[sl] = pv
    start_v[pl.ds(_NB, _L)] = jnp.broadcast_to(carry, (_L,))
    for j in range(_NB // _L):
        sl = pl.ds(j * _L, _L)
        base_v[sl] = base_v[sl] + start_v[sl]

    def pos_step(j, c):
        bv = bins_v[pl.ds(pl.multiple_of(j * _L, _L), _L)]
        pv = zero16
        for l in range(_L):
            b = bv[l]
            bb = base_v[pl.ds(b, _L)]
            pv = jnp.where(lane == l, bb[0], pv)
            base_v[pl.ds(b, _L)] = bb + oh16
        pos_v[j // 5, pl.ds((j % 5) * _L, _L)] = pv
        return c

    lax.fori_loop(0, _CHUNK // _L, pos_step, jnp.int32(0))
    for j in range(_CHUNK // 80):
        pj = pos_v.at[j]
        pltpu.sync_copy(xchunk_v.at[pl.ds(j * 80, 80)], xs_sh.at[pj])
        pltpu.sync_copy(origchunk_v.at[pl.ds(j * 80, 80)], orig_sh.at[pj])
    plsc.subcore_barrier()
    pltpu.sync_copy(xs_sh, xs_v.at[pl.ds(0, _PAD)])
    pltpu.sync_copy(orig_sh, orig_v.at[pl.ds(0, _PAD)])
    for j in range((_XPAD - _PAD) // _L):
        xs_v[pl.ds(_PAD + j * _L, _L)] = jnp.full((_L,), 9.0, jnp.float32)
        orig_v[pl.ds(_PAD + j * _L, _L)] = jnp.full((_L,), _BIGI, jnp.int32)

    for j in range(_QCHUNK // 80):
        for t in range(80 // _L):
            oidx_v[j, pl.ds(t * _L, _L)] = (
                orig_v[pl.ds(qbase + j * 80 + t * _L, _L)])

    inf16 = jnp.full((_L,), _INF, jnp.float32)
    big16 = jnp.full((_L,), _BIGI, jnp.int32)

    def group_step(g, gc):
        gq = pl.multiple_of(qbase + g * _L, _L)
        q = xs_v[pl.ds(gq, _L)]
        qi = orig_v[pl.ds(gq, _L)]
        xlo = q[0]
        xhi = q[_L - 1]
        lob = _bin_of(jnp.maximum(xlo - _MARGIN, 0.0))
        hib = _bin_of(jnp.maximum(xhi + _MARGIN, 0.0))
        c0 = start_v[pl.ds(lob, _L)][0]
        c1 = start_v[pl.ds(hib + 1, _L)][0]
        a0 = pl.multiple_of((c0 // _L) * _L, _L)
        nv = (c1 - a0 + (_L - 1)) // _L

        def cand_step(v, carry):
            m1, m2, m3, m4, i1, i2, i3, i4 = carry
            off = pl.multiple_of(a0 + v * _L, _L)
            xv = xs_v[pl.ds(off, _L)]
            ov = orig_v[pl.ds(off, _L)]
            for l in range(_L):
                xcb = jnp.broadcast_to(xv[l], (_L,))
                cib = jnp.broadcast_to(ov[l], (_L,))
                d = jnp.abs(q - xcb)
                dm = jnp.where((d > _RF) | (cib == qi), _INF, d)
                gt1 = (dm > m1) | ((dm == m1) & (cib > i1))
                gt2 = (dm > m2) | ((dm == m2) & (cib > i2))
                gt3 = (dm > m3) | ((dm == m3) & (cib > i3))
                gt4 = (dm > m4) | ((dm == m4) & (cib > i4))
                nm4 = jnp.where(gt4, m4, jnp.where(gt3, dm, m3))
                ni4 = jnp.where(gt4, i4, jnp.where(gt3, cib, i3))
                nm3 = jnp.where(gt3, m3, jnp.where(gt2, dm, m2))
                ni3 = jnp.where(gt3, i3, jnp.where(gt2, cib, i2))
                nm2 = jnp.where(gt2, m2, jnp.where(gt1, dm, m1))
                ni2 = jnp.where(gt2, i2, jnp.where(gt1, cib, i1))
                m1 = jnp.where(gt1, m1, dm)
                i1 = jnp.where(gt1, i1, cib)
                m2, m3, m4 = nm2, nm3, nm4
                i2, i3, i4 = ni2, ni3, ni4
            return m1, m2, m3, m4, i1, i2, i3, i4

        m1, m2, m3, m4, i1, i2, i3, i4 = lax.fori_loop(
            0, nv, cand_step,
            (inf16, inf16, inf16, inf16, big16, big16, big16, big16))

        gsl = pl.ds(pl.multiple_of(g * _L, _L), _L)
        acc = jnp.zeros((_L,), jnp.float32)
        for (mk, ik), znk_v in zip(
                ((m1, i1), (m2, i2), (m3, i3), (m4, i4)),
                (zn0_v, zn1_v, zn2_v, zn3_v)):
            valid = mk <= _RF
            znk_v[gsl] = jnp.where(valid, ik, qi)
            acc = acc + jnp.where(valid, mk * mk, 0.0)
        invs_v[gsl] = 1.0 / (acc + np.float32(1e-8))
        return gc

    lax.fori_loop(0, _QCHUNK // _L, group_step, jnp.int32(0))

    for j in range(_QCHUNK // 80):
        sl = pl.ds(j * 80, 80)
        oj = oidx_v.at[j]
        pltpu.sync_copy(zn0_v.at[sl], zn0_hbm.at[oj])
        pltpu.sync_copy(zn1_v.at[sl], zn1_hbm.at[oj])
        pltpu.sync_copy(zn2_v.at[sl], zn2_hbm.at[oj])
        pltpu.sync_copy(zn3_v.at[sl], zn3_hbm.at[oj])
        pltpu.sync_copy(invs_v.at[sl], invs_hbm.at[oj])


def kernel(x_f):
    x = x_f.reshape(-1)
    xp = jnp.full((_PAD,), 9.0, jnp.float32).at[:_N].set(x)
    mesh = plsc.VectorSubcoreMesh(
        core_axis_name="c", subcore_axis_name="s",
        num_cores=_NC, num_subcores=_NS)
    f = pl.kernel(
        _body,
        out_type=[jax.ShapeDtypeStruct((_PAD,), jnp.int32)] * 4
        + [jax.ShapeDtypeStruct((_PAD,), jnp.float32)],
        mesh=mesh,
        scratch_types=[
            pltpu.VMEM((_CHUNK,), jnp.float32),
            pltpu.VMEM((_CHUNK,), jnp.int32),
            pltpu.VMEM((_CHUNK,), jnp.int32),
            pltpu.VMEM((_NB + 16,), jnp.int32),
            pltpu.VMEM((_NS, _NB), jnp.int32),
            pltpu.VMEM((_NB,), jnp.int32),
            pltpu.VMEM((_NB + 32, ), jnp.int32),
            pltpu.VMEM((_NB + 16,), jnp.int32),
            pltpu.VMEM((_CHUNK // 80, 80), jnp.int32),
            pltpu.VMEM((_XPAD,), jnp.float32),
            pltpu.VMEM((_XPAD,), jnp.int32),
            pltpu.VMEM((_QCHUNK,), jnp.int32),
            pltpu.VMEM((_QCHUNK,), jnp.int32),
            pltpu.VMEM((_QCHUNK,), jnp.int32),
            pltpu.VMEM((_QCHUNK,), jnp.int32),
            pltpu.VMEM((_QCHUNK,), jnp.float32),
            pltpu.VMEM((_QCHUNK // 80, 80), jnp.int32),
            pltpu.VMEM_SHARED((_NS, _NB), jnp.int32),
            pltpu.VMEM_SHARED((_PAD,), jnp.float32),
            pltpu.VMEM_SHARED((_PAD,), jnp.int32),
        ],
    )
    zn0, zn1, zn2, zn3, invs = f(xp)
    zn = jnp.stack([zn0[:_N], zn1[:_N], zn2[:_N], zn3[:_N]], axis=1)
    return invs[:_N].reshape(_N, 1, 1), zn
